# baseline (speedup 1.0000x reference)
import jax
import jax.numpy as jnp
from jax import lax
from jax.experimental import pallas as pl
from jax.experimental.pallas import tpu as pltpu
from jax.experimental.pallas import tpu_sc as plsc

B = 4
S = 8192
D = 1024
L = 16
NC = 2
NS = 16
NW = NC * NS
S_PER_W = S // NW
K = 8
NCHUNK = S_PER_W // K
NPAIR = NCHUNK // 2
SEG = 4
SEGV = D // L // SEG


def _body(ids_hbm, tok_hbm, pos_hbm, out_hbm,
          idx_v, buf0, buf1, pos0, pos1,
          gsem0, gsem1, osem0, osem1, psem0, psem1, isem0, isem1):
    wid = lax.axis_index("s") * NC + lax.axis_index("c")
    base = wid * S_PER_W
    bufs = (buf0, buf1)
    poss = (pos0, pos1)
    gsems = (gsem0, gsem1)
    osems = (osem0, osem1)
    psems = (psem0, psem1)
    isems = (isem0, isem1)

    def add_chunk(jp):
        buf = bufs[jp]
        pos_ref = poss[jp]

        @plsc.parallel_loop(0, K * SEG)
        def _(i):
            r = i // SEG
            c0 = (i % SEG) * (SEGV * L)
            for l in range(SEGV):
                sl = pl.ds(c0 + l * L, L)
                pv = pos_ref[r, sl]
                for b in range(B):
                    plsc.addupdate(buf.at[b * K + r, sl], pv)

    def issue_gather(jp, ip):
        pltpu.async_copy(tok_hbm.at[idx_v.at[ip]], bufs[jp], gsems[jp])

    def wait_gather(jp):
        pltpu.make_async_copy(tok_hbm.at[idx_v.at[0]], bufs[jp],
                              gsems[jp]).wait()

    def issue_write(jp, b, off):
        pltpu.async_copy(bufs[jp].at[pl.ds(b * K, K)],
                         out_hbm.at[b, pl.ds(off, K)], osems[jp])

    def wait_writes(jp):
        for b in range(B):
            pltpu.make_async_copy(bufs[jp].at[pl.ds(b * K, K)],
                                  out_hbm.at[0, pl.ds(0, K)],
                                  osems[jp]).wait()

    def prefetch(jp, off):
        pltpu.async_copy(pos_hbm.at[pl.ds(off, K)], poss[jp], psems[jp])
        for b in range(B):
            pltpu.async_copy(ids_hbm.at[b, pl.ds(off, K)],
                             idx_v.at[jp, pl.ds(b * K, K)], isems[jp])

    def wait_pos(jp):
        pltpu.make_async_copy(pos_hbm.at[pl.ds(0, K)], poss[jp],
                              psems[jp]).wait()

    def wait_idx(jp):
        for b in range(B):
            pltpu.make_async_copy(ids_hbm.at[0, pl.ds(0, K)],
                                  idx_v.at[jp, pl.ds(b * K, K)],
                                  isems[jp]).wait()

    for b in range(B):
        pltpu.sync_copy(ids_hbm.at[b, pl.ds(base, K)],
                        idx_v.at[0, pl.ds(b * K, K)])
    pltpu.sync_copy(pos_hbm.at[pl.ds(base, K)], pos0)
    prefetch(1, base + K)
    issue_gather(0, 0)

    def pair(m, carry):
        for jj in range(2):
            j = 2 * m + jj
            off = base + j * K
            if jj == 0:
                @pl.when(m > 0)
                def _():
                    wait_writes(1)
                wait_idx(1)
                issue_gather(1, 1)
            else:
                wait_writes(0)

                @pl.when(m < NPAIR - 1)
                def _():
                    wait_idx(0)
                    issue_gather(0, 0)
            wait_gather(jj)
            if jj == 0:
                @pl.when(m > 0)
                def _():
                    wait_pos(0)
            else:
                wait_pos(1)
            add_chunk(jj)
            for b in range(B):
                issue_write(jj, b, off)
            @pl.when(m < NPAIR - 1)
            def _():
                prefetch(jj, off + 2 * K)
        return carry

    lax.fori_loop(0, NPAIR, pair, 0)
    wait_writes(1)


def kernel(input_ids, tok_table, pos_table):
    mesh = plsc.VectorSubcoreMesh(core_axis_name="c", subcore_axis_name="s")
    k = pl.kernel(
        _body,
        out_type=jax.ShapeDtypeStruct((B, S, D), jnp.float32),
        mesh=mesh,
        scratch_types=[
            pltpu.VMEM((2, B * K), jnp.int32),
            pltpu.VMEM((B * K, D), jnp.float32),
            pltpu.VMEM((B * K, D), jnp.float32),
            pltpu.VMEM((K, D), jnp.float32),
            pltpu.VMEM((K, D), jnp.float32),
            pltpu.SemaphoreType.DMA,
            pltpu.SemaphoreType.DMA,
            pltpu.SemaphoreType.DMA,
            pltpu.SemaphoreType.DMA,
            pltpu.SemaphoreType.DMA,
            pltpu.SemaphoreType.DMA,
            pltpu.SemaphoreType.DMA,
            pltpu.SemaphoreType.DMA,
        ],
    )
    return k(input_ids, tok_table, pos_table)


# speedup vs baseline: 1.0021x; 1.0021x over previous
import jax
import jax.numpy as jnp
from jax import lax
from jax.experimental import pallas as pl
from jax.experimental.pallas import tpu as pltpu
from jax.experimental.pallas import tpu_sc as plsc

B = 4
S = 8192
D = 1024
L = 16
NC = 2
NS = 16
NW = NC * NS
S_PER_W = S // NW
K = 8
NCHUNK = S_PER_W // K
NPAIR = NCHUNK // 2
SEG = 4
SEGV = D // L // SEG


def _body(ids_hbm, tok_hbm, pos_hbm, out_hbm,
          idx_v, buf0, buf1, pos0, pos1,
          gsem0, gsem1, osem0, osem1, psem0, psem1, isem0, isem1):
    wid = lax.axis_index("s") * NC + lax.axis_index("c")
    base = wid * S_PER_W
    bufs = (buf0, buf1)
    poss = (pos0, pos1)
    gsems = (gsem0, gsem1)
    osems = (osem0, osem1)
    psems = (psem0, psem1)
    isems = (isem0, isem1)

    def add_chunk(jp):
        buf = bufs[jp]
        pos_ref = poss[jp]

        @plsc.parallel_loop(0, K * SEG)
        def _(i):
            r = i // SEG
            c0 = (i % SEG) * (SEGV * L)
            for l in range(SEGV):
                sl = pl.ds(c0 + l * L, L)
                pv = pos_ref[r, sl]
                for b in range(B):
                    plsc.addupdate(buf.at[b * K + r, sl], pv)

    def issue_gather(jp, ip):
        pltpu.async_copy(tok_hbm.at[idx_v.at[ip]], bufs[jp], gsems[jp])

    def wait_gather(jp):
        pltpu.make_async_copy(tok_hbm.at[idx_v.at[0]], bufs[jp],
                              gsems[jp]).wait()

    def issue_write(jp, b, off):
        pltpu.async_copy(bufs[jp].at[pl.ds(b * K, K)],
                         out_hbm.at[b, pl.ds(off, K)], osems[jp])

    def wait_writes(jp):
        for b in range(B):
            pltpu.make_async_copy(bufs[jp].at[pl.ds(b * K, K)],
                                  out_hbm.at[0, pl.ds(0, K)],
                                  osems[jp]).wait()

    def prefetch(jp, off):
        pltpu.async_copy(pos_hbm.at[pl.ds(off, K)], poss[jp], psems[jp])
        for b in range(B):
            pltpu.async_copy(ids_hbm.at[b, pl.ds(off, K)],
                             idx_v.at[jp, pl.ds(b * K, K)], isems[jp])

    def wait_pos(jp):
        pltpu.make_async_copy(pos_hbm.at[pl.ds(0, K)], poss[jp],
                              psems[jp]).wait()

    def wait_idx(jp):
        for b in range(B):
            pltpu.make_async_copy(ids_hbm.at[0, pl.ds(0, K)],
                                  idx_v.at[jp, pl.ds(b * K, K)],
                                  isems[jp]).wait()

    for b in range(B):
        pltpu.sync_copy(ids_hbm.at[b, pl.ds(base, K)],
                        idx_v.at[0, pl.ds(b * K, K)])
    pltpu.sync_copy(pos_hbm.at[pl.ds(base, K)], pos0)
    prefetch(1, base + K)
    issue_gather(0, 0)

    def pair(m, carry):
        for jj in range(2):
            j = 2 * m + jj
            off = base + j * K
            wait_gather(jj)
            if jj == 0:
                @pl.when(m > 0)
                def _():
                    wait_pos(0)
            else:
                wait_pos(1)
            if jj == 0:
                @pl.when(m > 0)
                def _():
                    wait_writes(1)
                wait_idx(1)
                issue_gather(1, 1)
            else:
                wait_writes(0)

                @pl.when(m < NPAIR - 1)
                def _():
                    wait_idx(0)
                    issue_gather(0, 0)
            add_chunk(jj)
            for b in range(B):
                issue_write(jj, b, off)
            @pl.when(m < NPAIR - 1)
            def _():
                prefetch(jj, off + 2 * K)
        return carry

    lax.fori_loop(0, NPAIR, pair, 0)
    wait_writes(1)


def kernel(input_ids, tok_table, pos_table):
    mesh = plsc.VectorSubcoreMesh(core_axis_name="c", subcore_axis_name="s")
    k = pl.kernel(
        _body,
        out_type=jax.ShapeDtypeStruct((B, S, D), jnp.float32),
        mesh=mesh,
        scratch_types=[
            pltpu.VMEM((2, B * K), jnp.int32),
            pltpu.VMEM((B * K, D), jnp.float32),
            pltpu.VMEM((B * K, D), jnp.float32),
            pltpu.VMEM((K, D), jnp.float32),
            pltpu.VMEM((K, D), jnp.float32),
            pltpu.SemaphoreType.DMA,
            pltpu.SemaphoreType.DMA,
            pltpu.SemaphoreType.DMA,
            pltpu.SemaphoreType.DMA,
            pltpu.SemaphoreType.DMA,
            pltpu.SemaphoreType.DMA,
            pltpu.SemaphoreType.DMA,
            pltpu.SemaphoreType.DMA,
        ],
    )
    return k(input_ids, tok_table, pos_table)


# speedup vs baseline: 1.0199x; 1.0177x over previous
import jax
import jax.numpy as jnp
from jax import lax
from jax.experimental import pallas as pl
from jax.experimental.pallas import tpu as pltpu
from jax.experimental.pallas import tpu_sc as plsc

B = 4
S = 8192
D = 1024
L = 16
NC = 2
NS = 16
NW = NC * NS
S_PER_W = S // NW
K = 8
NCHUNK = S_PER_W // K
NPAIR = NCHUNK // 2
SEG = 4
SEGV = D // L // SEG


def _body(ids_hbm, tok_hbm, pos_hbm, out_hbm,
          idx_v, buf0, buf1, pos0, pos1,
          gsem0, gsem1, osem0, osem1, psem0, psem1, isem0, isem1):
    wid = lax.axis_index("s") * NC + lax.axis_index("c")
    base = wid * S_PER_W
    bufs = (buf0, buf1)
    poss = (pos0, pos1)
    gsems = (gsem0, gsem1)
    osems = (osem0, osem1)
    psems = (psem0, psem1)
    isems = (isem0, isem1)

    def add_chunk(jp):
        buf = bufs[jp]
        pos_ref = poss[jp]

        @plsc.parallel_loop(0, K * SEG)
        def _(i):
            r = i // SEG
            c0 = (i % SEG) * (SEGV * L)
            for l in range(SEGV):
                sl = pl.ds(c0 + l * L, L)
                pv = pos_ref[r, sl]
                for b in range(B):
                    plsc.addupdate(buf.at[b * K + r, sl], pv)

    def issue_gather(jp, ip):
        pltpu.async_copy(tok_hbm.at[idx_v.at[ip]], bufs[jp], gsems[jp])

    def wait_gather(jp):
        pltpu.make_async_copy(tok_hbm.at[idx_v.at[0]], bufs[jp],
                              gsems[jp]).wait()

    def issue_write(jp, b, off):
        pltpu.async_copy(bufs[jp].at[pl.ds(b * K, K)],
                         out_hbm.at[b, pl.ds(off, K)], osems[jp])

    def wait_writes(jp):
        for b in range(B):
            pltpu.make_async_copy(bufs[jp].at[pl.ds(b * K, K)],
                                  out_hbm.at[0, pl.ds(0, K)],
                                  osems[jp]).wait()

    def prefetch(jp, off):
        pltpu.async_copy(pos_hbm.at[pl.ds(off, K)], poss[jp], psems[jp])
        for b in range(B):
            pltpu.async_copy(ids_hbm.at[b, pl.ds(off, K)],
                             idx_v.at[jp, pl.ds(b * K, K)], isems[jp])

    def wait_pos(jp):
        pltpu.make_async_copy(pos_hbm.at[pl.ds(0, K)], poss[jp],
                              psems[jp]).wait()

    def wait_idx(jp):
        for b in range(B):
            pltpu.make_async_copy(ids_hbm.at[0, pl.ds(0, K)],
                                  idx_v.at[jp, pl.ds(b * K, K)],
                                  isems[jp]).wait()

    prefetch(0, base)
    prefetch(1, base + K)
    wait_idx(0)
    issue_gather(0, 0)

    def pair(m, carry):
        for jj in range(2):
            j = 2 * m + jj
            off = base + j * K
            wait_gather(jj)
            wait_pos(jj)
            if jj == 0:
                @pl.when(m > 0)
                def _():
                    wait_writes(1)
                wait_idx(1)
                issue_gather(1, 1)
            else:
                wait_writes(0)

                @pl.when(m < NPAIR - 1)
                def _():
                    wait_idx(0)
                    issue_gather(0, 0)
            add_chunk(jj)
            for b in range(B):
                issue_write(jj, b, off)
            @pl.when(m < NPAIR - 1)
            def _():
                prefetch(jj, off + 2 * K)
        return carry

    lax.fori_loop(0, NPAIR, pair, 0)
    wait_writes(1)


def kernel(input_ids, tok_table, pos_table):
    mesh = plsc.VectorSubcoreMesh(core_axis_name="c", subcore_axis_name="s")
    k = pl.kernel(
        _body,
        out_type=jax.ShapeDtypeStruct((B, S, D), jnp.float32),
        mesh=mesh,
        scratch_types=[
            pltpu.VMEM((2, B * K), jnp.int32),
            pltpu.VMEM((B * K, D), jnp.float32),
            pltpu.VMEM((B * K, D), jnp.float32),
            pltpu.VMEM((K, D), jnp.float32),
            pltpu.VMEM((K, D), jnp.float32),
            pltpu.SemaphoreType.DMA,
            pltpu.SemaphoreType.DMA,
            pltpu.SemaphoreType.DMA,
            pltpu.SemaphoreType.DMA,
            pltpu.SemaphoreType.DMA,
            pltpu.SemaphoreType.DMA,
            pltpu.SemaphoreType.DMA,
            pltpu.SemaphoreType.DMA,
        ],
    )
    return k(input_ids, tok_table, pos_table)
